# baseline (speedup 1.0000x reference)
import functools

import numpy as np
import jax
import jax.numpy as jnp
from jax import lax
from jax.experimental import pallas as pl
from jax.experimental.pallas import tpu as pltpu
from jax.experimental.pallas import tpu_sc as plsc

OUT = 7
C = 256
CW = 128
NBOX = 1000
NBOX_PAD = 1024
TABLE_ROWS = 16384 + 4096 + 1024 + 256
NC, NS = 2, 16
NW = NC * NS
BPW = NBOX_PAD // NW
NSLOT = 4

_T1 = float((224.0 * 0.5 - 224e-6) ** 2)
_T2 = float((224.0 * 1.0 - 224e-6) ** 2)
_T3 = float((224.0 * 2.0 - 224e-6) ** 2)
_BASES = (0.0, 16384.0, 20480.0, 21504.0)


def _lane_consts():
    t = lax.iota(jnp.int32, 16)
    sy = ((t >> 3) & 1).astype(jnp.float32)
    sx = ((t >> 2) & 1).astype(jnp.float32)
    dyi = (t >> 1) & 1
    dxi = t & 1
    c_oy = (sy + 0.5) / 2.0
    c_ox = (sx + 0.5) / 2.0
    return (c_oy, c_ox, dyi.astype(jnp.float32), dxi.astype(jnp.float32),
            dyi > 0, dxi > 0)


def _splat(ref, i):
    return plsc.load_gather(ref, [jnp.full((16,), i, jnp.int32)])


def _sc_body(table, boxes_t, out, bxv, prm, idxb, wb, taps, stage, sem, osem):
    wid = lax.axis_index("s") * NC + lax.axis_index("c")
    b0 = wid * BPW

    c_oy, c_ox, dyv, dxv, dym, dxm = _lane_consts()

    for k in range(4):
        pltpu.sync_copy(boxes_t.at[pl.ds(k * NBOX_PAD + b0, BPW)],
                        bxv.at[pl.ds(k * BPW, BPW)])

    for g in range(2):
        x1 = bxv[pl.ds(0 * BPW + g * 16, 16)]
        y1 = bxv[pl.ds(1 * BPW + g * 16, 16)]
        x2 = bxv[pl.ds(2 * BPW + g * 16, 16)]
        y2 = bxv[pl.ds(3 * BPW + g * 16, 16)]
        area = jnp.maximum((x2 - x1) * (y2 - y1), 1e-6)
        l1 = area >= _T1
        l2 = area >= _T2
        l3 = area >= _T3
        scale = jnp.where(l1, jnp.where(l2, jnp.where(l3, 0.03125, 0.0625), 0.125), 0.25)
        basef = jnp.where(l1, jnp.where(l2, jnp.where(l3, _BASES[3], _BASES[2]), _BASES[1]), _BASES[0])
        wf = 512.0 * scale
        px1 = x1 * scale
        py1 = y1 * scale
        binw = jnp.maximum(x2 * scale - px1, 1.0) / 7.0
        binh = jnp.maximum(y2 * scale - py1, 1.0) / 7.0
        for p, v in enumerate((px1, py1, binw, binh, basef, wf)):
            prm[pl.ds(16 + p * BPW + g * 16, 16)] = v

    NSTEP = BPW * OUT

    def fill_step(b, i, p):
        px1s = _splat(prm, 16 + 0 * BPW + b)
        py1s = _splat(prm, 16 + 1 * BPW + b)
        binws = _splat(prm, 16 + 2 * BPW + b)
        binhs = _splat(prm, 16 + 3 * BPW + b)
        basefs = _splat(prm, 16 + 4 * BPW + b)
        wfs = _splat(prm, 16 + 5 * BPW + b)
        hm1 = wfs - 1.0
        fiv = jnp.full((16,), i, jnp.int32).astype(jnp.float32)
        ys = py1s + (fiv + c_oy) * binhs
        vy = ys <= wfs
        yc = jnp.minimum(ys, hm1)
        y0 = yc.astype(jnp.int32).astype(jnp.float32)
        ly = yc - y0
        wy = jnp.where(dym, ly, 1.0 - ly)
        yb = basefs + jnp.minimum(y0 + dyv, hm1) * wfs
        for j in range(OUT):
            xs = px1s + (np.float32(j) + c_ox) * binws
            vx = xs <= wfs
            xc = jnp.minimum(xs, hm1)
            x0 = xc.astype(jnp.int32).astype(jnp.float32)
            lx = xc - x0
            wx = jnp.where(dxm, lx, 1.0 - lx)
            xi = jnp.minimum(x0 + dxv, hm1)
            idxb[p, pl.ds(j * 16, 16)] = (yb + xi).astype(jnp.int32)
            wb[p, pl.ds(16 + j * 16, 16)] = jnp.where(vy & vx, wy * wx * 0.25, 0.0)
        pltpu.async_copy(table.at[idxb.at[p]], taps.at[p], sem.at[p])

    def out_off(b, i):
        return (b0 + b) * (49 * C) + i * (OUT * C)

    for kk in range(NSLOT - 1):
        fill_step(0, kk, kk)

    def body(k, carry):
        b, i = carry
        p = k % NSLOT
        bn = jnp.where(i == OUT - 1, b + 1, b)
        inx = jnp.where(i == OUT - 1, 0, i + 1)
        i3r = i + (NSLOT - 1)
        b3 = jnp.where(i3r >= OUT, b + 1, b)
        i3 = jnp.where(i3r >= OUT, i3r - OUT, i3r)

        @pl.when(k + (NSLOT - 1) < NSTEP)
        def _():
            fill_step(b3, i3, (k + (NSLOT - 1)) % NSLOT)

        @pl.when(k >= 2)
        def _():
            bo = jnp.where(i <= 1, b - 1, b)
            io = jnp.where(i <= 1, i + (OUT - 2), i - 2)
            pltpu.make_async_copy(
                stage.at[k % 2], out.at[pl.ds(out_off(bo, io), OUT * C)],
                osem.at[k % 2]).wait()

        pltpu.make_async_copy(table.at[idxb.at[p]], taps.at[p], sem.at[p]).wait()
        for j in range(OUT):
            acc_lo = [None] * 8
            acc_hi = [None] * 8
            for t in range(16):
                wspl = plsc.load_gather(
                    wb, [jnp.full((16,), p, jnp.int32),
                         jnp.full((16,), 16 + j * 16 + t, jnp.int32)])
                r = j * 16 + t
                for c in range(8):
                    vi = plsc.bitcast(taps[p, r, pl.ds(c * 16, 16)], jnp.int32)
                    lo = plsc.bitcast(vi << 16, jnp.float32)
                    hi = plsc.bitcast(vi & jnp.int32(-65536), jnp.float32)
                    tlo = wspl * lo
                    thi = wspl * hi
                    acc_lo[c] = tlo if t == 0 else acc_lo[c] + tlo
                    acc_hi[c] = thi if t == 0 else acc_hi[c] + thi
            for c in range(8):
                stage[k % 2, pl.ds(j * C + c * 16, 16)] = acc_lo[c]
                stage[k % 2, pl.ds(j * C + CW + c * 16, 16)] = acc_hi[c]
        pltpu.async_copy(stage.at[k % 2], out.at[pl.ds(out_off(b, i), OUT * C)],
                         osem.at[k % 2])
        return (bn, inx)

    lax.fori_loop(0, NSTEP, body, (jnp.int32(0), jnp.int32(0)))

    for k in (NSTEP - 2, NSTEP - 1):
        b, i = k // OUT, k % OUT
        pltpu.make_async_copy(
            stage.at[k % 2], out.at[pl.ds(out_off(b, i), OUT * C)],
            osem.at[k % 2]).wait()


@jax.jit
def _pooler(table, boxes_t):
    mesh = plsc.VectorSubcoreMesh(core_axis_name="c", subcore_axis_name="s")
    f = functools.partial(
        pl.kernel,
        out_type=jax.ShapeDtypeStruct((NBOX_PAD * 49 * C,), jnp.float32),
        mesh=mesh,
        compiler_params=pltpu.CompilerParams(needs_layout_passes=False),
        scratch_types=[
            pltpu.VMEM((4 * BPW,), jnp.float32),
            pltpu.VMEM((16 + 6 * BPW,), jnp.float32),
            pltpu.VMEM((NSLOT, OUT * 16), jnp.int32),
            pltpu.VMEM((NSLOT, 16 + OUT * 16), jnp.float32),
            pltpu.VMEM((NSLOT, OUT * 16, CW), jnp.float32),
            pltpu.VMEM((2, OUT * C), jnp.float32),
            pltpu.SemaphoreType.DMA((NSLOT,)),
            pltpu.SemaphoreType.DMA((2,)),
        ],
    )(_sc_body)
    return f(table, boxes_t)


def kernel(feat0, feat1, feat2, feat3, boxes):
    feats = (feat0[0], feat1[0], feat2[0], feat3[0])
    table = jnp.concatenate(
        [jnp.transpose(f, (1, 2, 0)).reshape(-1, C) for f in feats], axis=0)
    packed = lax.bitcast_convert_type(
        table.astype(jnp.bfloat16).reshape(TABLE_ROWS, CW, 2), jnp.float32)
    boxes_t = jnp.zeros((4, NBOX_PAD), jnp.float32).at[:, :NBOX].set(boxes.T).reshape(-1)
    out = _pooler(packed, boxes_t)
    res = out[:NBOX * 49 * C]
    return res.reshape(NBOX, C, OUT, OUT)


# speedup vs baseline: 1.3836x; 1.3836x over previous
import functools

import numpy as np
import jax
import jax.numpy as jnp
from jax import lax
from jax.experimental import pallas as pl
from jax.experimental.pallas import tpu as pltpu
from jax.experimental.pallas import tpu_sc as plsc

OUT = 7
C = 256
CW = 128
NBOX = 1000
NBOX_PAD = 1024
TABLE_ROWS = 16384 + 4096 + 1024 + 256
NC, NS = 2, 16
NW = NC * NS
BPW = NBOX_PAD // NW
NSLOT = 4
RPS = OUT * 16
OFFW = 160

_T1 = float((224.0 * 0.5 - 224e-6) ** 2)
_T2 = float((224.0 * 1.0 - 224e-6) ** 2)
_T3 = float((224.0 * 2.0 - 224e-6) ** 2)
_BASES = (0.0, 16384.0, 20480.0, 21504.0)


def _lane_consts():
    t = lax.iota(jnp.int32, 16)
    syi = (t >> 3) & 1
    sxi = (t >> 2) & 1
    dyi = (t >> 1) & 1
    dxi = t & 1
    c_oy = (syi.astype(jnp.float32) + 0.5) / 2.0
    c_ox = (sxi.astype(jnp.float32) + 0.5) / 2.0
    s2d = syi * 2 + dyi
    return (c_oy, c_ox, dyi.astype(jnp.float32), dxi.astype(jnp.float32),
            dyi > 0, dxi > 0, s2d)


def _splat(ref, i):
    return plsc.load_gather(ref, [jnp.full((16,), i, jnp.int32)])


def _sc_body(table, boxes_t, out, bxv, prm, idxb, idxf, offb, wb, taps, stage,
             sem, osem):
    wid = lax.axis_index("s") * NC + lax.axis_index("c")
    b0 = wid * BPW

    c_oy, c_ox, dyv, dxv, dym, dxm, s2d = _lane_consts()
    iot = lax.iota(jnp.int32, 16)

    for k in range(4):
        pltpu.sync_copy(boxes_t.at[pl.ds(k * NBOX_PAD + b0, BPW)],
                        bxv.at[pl.ds(k * BPW, BPW)])

    for g in range(2):
        x1 = bxv[pl.ds(0 * BPW + g * 16, 16)]
        y1 = bxv[pl.ds(1 * BPW + g * 16, 16)]
        x2 = bxv[pl.ds(2 * BPW + g * 16, 16)]
        y2 = bxv[pl.ds(3 * BPW + g * 16, 16)]
        area = jnp.maximum((x2 - x1) * (y2 - y1), 1e-6)
        l1 = area >= _T1
        l2 = area >= _T2
        l3 = area >= _T3
        scale = jnp.where(l1, jnp.where(l2, jnp.where(l3, 0.03125, 0.0625), 0.125), 0.25)
        basef = jnp.where(l1, jnp.where(l2, jnp.where(l3, _BASES[3], _BASES[2]), _BASES[1]), _BASES[0])
        wf = 512.0 * scale
        px1 = x1 * scale
        py1 = y1 * scale
        binw = jnp.maximum(x2 * scale - px1, 1.0) / 7.0
        binh = jnp.maximum(y2 * scale - py1, 1.0) / 7.0
        xs_min = px1 + 0.25 * binw
        xs_max = px1 + 6.75 * binw
        xlo = jnp.minimum(xs_min.astype(jnp.int32).astype(jnp.float32), wf - 16.0)
        xc_max = jnp.minimum(xs_max, wf - 1.0)
        tapmax = jnp.minimum(xc_max.astype(jnp.int32).astype(jnp.float32) + 1.0,
                             wf - 1.0)
        fastf = jnp.where(tapmax - xlo <= 15.0, 1.0, 0.0)
        for p, v in enumerate((px1, py1, binw, binh, basef, wf, xlo, fastf)):
            prm[pl.ds(16 + p * BPW + g * 16, 16)] = v

    NSTEP = BPW * OUT

    def is_fast(b):
        return _splat(prm, 16 + 7 * BPW + b)[0] > 0.5

    def fill_step(b, i, p):
        px1s = _splat(prm, 16 + 0 * BPW + b)
        py1s = _splat(prm, 16 + 1 * BPW + b)
        binws = _splat(prm, 16 + 2 * BPW + b)
        binhs = _splat(prm, 16 + 3 * BPW + b)
        basefs = _splat(prm, 16 + 4 * BPW + b)
        wfs = _splat(prm, 16 + 5 * BPW + b)
        xlos = _splat(prm, 16 + 6 * BPW + b)
        fasts = _splat(prm, 16 + 7 * BPW + b)
        fastm = fasts > 0.5
        xloi = xlos.astype(jnp.int32)
        hm1 = wfs - 1.0
        fiv = jnp.full((16,), i, jnp.int32).astype(jnp.float32)
        ys = py1s + (fiv + c_oy) * binhs
        vy = ys <= wfs
        yc = jnp.minimum(ys, hm1)
        y0 = yc.astype(jnp.int32).astype(jnp.float32)
        ly = yc - y0
        wy = jnp.where(dym, ly, 1.0 - ly)
        yb = basefs + jnp.minimum(y0 + dyv, hm1) * wfs
        ybi = yb.astype(jnp.int32)
        offb[p, pl.ds(OFFW - 16, 16)] = ybi
        pofs = jnp.full((16,), p * RPS, jnp.int32)
        for j in range(OUT):
            xs = px1s + (np.float32(j) + c_ox) * binws
            vx = xs <= wfs
            xc = jnp.minimum(xs, hm1)
            x0 = xc.astype(jnp.int32).astype(jnp.float32)
            lx = xc - x0
            wx = jnp.where(dxm, lx, 1.0 - lx)
            xi = jnp.minimum(x0 + dxv, hm1)
            xii = xi.astype(jnp.int32)
            idxb[p, pl.ds(j * 16, 16)] = ybi + xii
            off = jnp.where(fastm, s2d * 16 + (xii - xloi),
                            jnp.full((16,), j * 16, jnp.int32) + iot)
            offb[p, pl.ds(16 + j * 16, 16)] = pofs + off
            wb[p, pl.ds(16 + j * 16, 16)] = jnp.where(vy & vx, wy * wx * 0.25, 0.0)
        for r, lane in enumerate((0, 2, 8, 10)):
            yrow = plsc.load_gather(
                offb, [jnp.full((16,), p, jnp.int32),
                       jnp.full((16,), OFFW - 16 + lane, jnp.int32)])
            idxf[p, pl.ds(r * 16, 16)] = yrow + xloi + iot
        fc = is_fast(b)

        @pl.when(fc)
        def _():
            pltpu.async_copy(table.at[idxf.at[p]],
                             taps.at[pl.ds(p * RPS, 64)], sem.at[p])

        @pl.when(jnp.logical_not(fc))
        def _():
            pltpu.async_copy(table.at[idxb.at[p]],
                             taps.at[pl.ds(p * RPS, RPS)], sem.at[p])

    def out_off(b, i):
        return (b0 + b) * (49 * C) + i * (OUT * C)

    for kk in range(NSLOT - 1):
        fill_step(0, kk, kk)

    def body(k, carry):
        b, i = carry
        p = k % NSLOT
        bn = jnp.where(i == OUT - 1, b + 1, b)
        inx = jnp.where(i == OUT - 1, 0, i + 1)
        i3r = i + (NSLOT - 1)
        b3 = jnp.where(i3r >= OUT, b + 1, b)
        i3 = jnp.where(i3r >= OUT, i3r - OUT, i3r)

        @pl.when(k + (NSLOT - 1) < NSTEP)
        def _():
            fill_step(b3, i3, (k + (NSLOT - 1)) % NSLOT)

        @pl.when(k >= 2)
        def _():
            bo = jnp.where(i <= 1, b - 1, b)
            io = jnp.where(i <= 1, i + (OUT - 2), i - 2)
            pltpu.make_async_copy(
                stage.at[k % 2], out.at[pl.ds(out_off(bo, io), OUT * C)],
                osem.at[k % 2]).wait()

        fc = is_fast(b)

        @pl.when(fc)
        def _():
            pltpu.make_async_copy(table.at[idxf.at[p]],
                                  taps.at[pl.ds(p * RPS, 64)], sem.at[p]).wait()

        @pl.when(jnp.logical_not(fc))
        def _():
            pltpu.make_async_copy(table.at[idxb.at[p]],
                                  taps.at[pl.ds(p * RPS, RPS)], sem.at[p]).wait()

        cvecs = [c * 16 + iot for c in range(8)]
        for j in range(OUT):
            acc_lo = [None] * 8
            acc_hi = [None] * 8
            for t in range(16):
                wspl = plsc.load_gather(
                    wb, [jnp.full((16,), p, jnp.int32),
                         jnp.full((16,), 16 + j * 16 + t, jnp.int32)])
                rospl = plsc.load_gather(
                    offb, [jnp.full((16,), p, jnp.int32),
                           jnp.full((16,), 16 + j * 16 + t, jnp.int32)])
                for c in range(8):
                    row = plsc.load_gather(taps, [rospl, cvecs[c]])
                    vi = plsc.bitcast(row, jnp.int32)
                    lo = plsc.bitcast(vi << 16, jnp.float32)
                    hi = plsc.bitcast(vi & jnp.int32(-65536), jnp.float32)
                    tlo = wspl * lo
                    thi = wspl * hi
                    acc_lo[c] = tlo if t == 0 else acc_lo[c] + tlo
                    acc_hi[c] = thi if t == 0 else acc_hi[c] + thi
            for c in range(8):
                stage[k % 2, pl.ds(j * C + c * 16, 16)] = acc_lo[c]
                stage[k % 2, pl.ds(j * C + CW + c * 16, 16)] = acc_hi[c]
        pltpu.async_copy(stage.at[k % 2], out.at[pl.ds(out_off(b, i), OUT * C)],
                         osem.at[k % 2])
        return (bn, inx)

    lax.fori_loop(0, NSTEP, body, (jnp.int32(0), jnp.int32(0)))

    for k in (NSTEP - 2, NSTEP - 1):
        b, i = k // OUT, k % OUT
        pltpu.make_async_copy(
            stage.at[k % 2], out.at[pl.ds(out_off(b, i), OUT * C)],
            osem.at[k % 2]).wait()


@jax.jit
def _pooler(table, boxes_t):
    mesh = plsc.VectorSubcoreMesh(core_axis_name="c", subcore_axis_name="s")
    f = functools.partial(
        pl.kernel,
        out_type=jax.ShapeDtypeStruct((NBOX_PAD * 49 * C,), jnp.float32),
        mesh=mesh,
        compiler_params=pltpu.CompilerParams(needs_layout_passes=False),
        scratch_types=[
            pltpu.VMEM((4 * BPW,), jnp.float32),
            pltpu.VMEM((16 + 8 * BPW,), jnp.float32),
            pltpu.VMEM((NSLOT, RPS), jnp.int32),
            pltpu.VMEM((NSLOT, 64), jnp.int32),
            pltpu.VMEM((NSLOT, OFFW), jnp.int32),
            pltpu.VMEM((NSLOT, 16 + RPS), jnp.float32),
            pltpu.VMEM((NSLOT * RPS, CW), jnp.float32),
            pltpu.VMEM((2, OUT * C), jnp.float32),
            pltpu.SemaphoreType.DMA((NSLOT,)),
            pltpu.SemaphoreType.DMA((2,)),
        ],
    )(_sc_body)
    return f(table, boxes_t)


def kernel(feat0, feat1, feat2, feat3, boxes):
    feats = (feat0[0], feat1[0], feat2[0], feat3[0])
    table = jnp.concatenate(
        [jnp.transpose(f, (1, 2, 0)).reshape(-1, C) for f in feats], axis=0)
    packed = lax.bitcast_convert_type(
        table.astype(jnp.bfloat16).reshape(TABLE_ROWS, CW, 2), jnp.float32)
    boxes_t = jnp.zeros((4, NBOX_PAD), jnp.float32).at[:, :NBOX].set(boxes.T).reshape(-1)
    out = _pooler(packed, boxes_t)
    res = out.reshape(NBOX_PAD, 49, 2, CW)[:NBOX]
    res = jnp.transpose(res, (0, 3, 2, 1)).reshape(NBOX, C, 49)
    return res.reshape(NBOX, C, OUT, OUT)


# speedup vs baseline: 1.6153x; 1.1675x over previous
import functools

import numpy as np
import jax
import jax.numpy as jnp
from jax import lax
from jax.experimental import pallas as pl
from jax.experimental.pallas import tpu as pltpu
from jax.experimental.pallas import tpu_sc as plsc

OUT = 7
C = 256
CW = 128
NBOX = 1000
NBOX_PAD = 1024
TABLE_ROWS = 16384 + 4096 + 1024 + 256
NC, NS = 2, 16
NW = NC * NS
BPW = NBOX_PAD // NW
NSLOT = 4

_T1 = float((224.0 * 0.5 - 224e-6) ** 2)
_T2 = float((224.0 * 1.0 - 224e-6) ** 2)
_T3 = float((224.0 * 2.0 - 224e-6) ** 2)
_BASES = (0.0, 16384.0, 20480.0, 21504.0)


def _lane_consts():
    t = lax.iota(jnp.int32, 16)
    sy = ((t >> 3) & 1).astype(jnp.float32)
    sx = ((t >> 2) & 1).astype(jnp.float32)
    dyi = (t >> 1) & 1
    dxi = t & 1
    c_oy = (sy + 0.5) / 2.0
    c_ox = (sx + 0.5) / 2.0
    return (c_oy, c_ox, dyi.astype(jnp.float32), dxi.astype(jnp.float32),
            dyi > 0, dxi > 0)


def _splat(ref, i):
    return plsc.load_gather(ref, [jnp.full((16,), i, jnp.int32)])


def _sc_body(table, boxes_t, out, bxv, prm, idxb, wb, taps, stage, sem, osem):
    wid = lax.axis_index("s") * NC + lax.axis_index("c")
    b0 = wid * BPW

    c_oy, c_ox, dyv, dxv, dym, dxm = _lane_consts()

    for k in range(4):
        pltpu.sync_copy(boxes_t.at[pl.ds(k * NBOX_PAD + b0, BPW)],
                        bxv.at[pl.ds(k * BPW, BPW)])

    for g in range(2):
        x1 = bxv[pl.ds(0 * BPW + g * 16, 16)]
        y1 = bxv[pl.ds(1 * BPW + g * 16, 16)]
        x2 = bxv[pl.ds(2 * BPW + g * 16, 16)]
        y2 = bxv[pl.ds(3 * BPW + g * 16, 16)]
        area = jnp.maximum((x2 - x1) * (y2 - y1), 1e-6)
        l1 = area >= _T1
        l2 = area >= _T2
        l3 = area >= _T3
        scale = jnp.where(l1, jnp.where(l2, jnp.where(l3, 0.03125, 0.0625), 0.125), 0.25)
        basef = jnp.where(l1, jnp.where(l2, jnp.where(l3, _BASES[3], _BASES[2]), _BASES[1]), _BASES[0])
        wf = 512.0 * scale
        px1 = x1 * scale
        py1 = y1 * scale
        binw = jnp.maximum(x2 * scale - px1, 1.0) / 7.0
        binh = jnp.maximum(y2 * scale - py1, 1.0) / 7.0
        for p, v in enumerate((px1, py1, binw, binh, basef, wf)):
            prm[pl.ds(16 + p * BPW + g * 16, 16)] = v

    NSTEP = BPW * OUT

    def fill_step(b, i, p):
        px1s = _splat(prm, 16 + 0 * BPW + b)
        py1s = _splat(prm, 16 + 1 * BPW + b)
        binws = _splat(prm, 16 + 2 * BPW + b)
        binhs = _splat(prm, 16 + 3 * BPW + b)
        basefs = _splat(prm, 16 + 4 * BPW + b)
        wfs = _splat(prm, 16 + 5 * BPW + b)
        hm1 = wfs - 1.0
        fiv = jnp.full((16,), i, jnp.int32).astype(jnp.float32)
        ys = py1s + (fiv + c_oy) * binhs
        vy = ys <= wfs
        yc = jnp.minimum(ys, hm1)
        y0 = yc.astype(jnp.int32).astype(jnp.float32)
        ly = yc - y0
        wy = jnp.where(dym, ly, 1.0 - ly)
        yb = basefs + jnp.minimum(y0 + dyv, hm1) * wfs
        for j in range(OUT):
            xs = px1s + (np.float32(j) + c_ox) * binws
            vx = xs <= wfs
            xc = jnp.minimum(xs, hm1)
            x0 = xc.astype(jnp.int32).astype(jnp.float32)
            lx = xc - x0
            wx = jnp.where(dxm, lx, 1.0 - lx)
            xi = jnp.minimum(x0 + dxv, hm1)
            idxb[p, pl.ds(j * 16, 16)] = (yb + xi).astype(jnp.int32)
            wb[p, pl.ds(16 + j * 16, 16)] = jnp.where(vy & vx, wy * wx * 0.25, 0.0)
        pltpu.async_copy(table.at[idxb.at[p]], taps.at[p], sem.at[p])

    def out_off(b, i):
        return (b0 + b) * (49 * C) + i * (OUT * C)

    for kk in range(NSLOT - 1):
        fill_step(0, kk, kk)

    def body(k, carry):
        b, i = carry
        p = k % NSLOT
        bn = jnp.where(i == OUT - 1, b + 1, b)
        inx = jnp.where(i == OUT - 1, 0, i + 1)
        i3r = i + (NSLOT - 1)
        b3 = jnp.where(i3r >= OUT, b + 1, b)
        i3 = jnp.where(i3r >= OUT, i3r - OUT, i3r)

        @pl.when(k + (NSLOT - 1) < NSTEP)
        def _():
            fill_step(b3, i3, (k + (NSLOT - 1)) % NSLOT)

        @pl.when(k >= 2)
        def _():
            bo = jnp.where(i <= 1, b - 1, b)
            io = jnp.where(i <= 1, i + (OUT - 2), i - 2)
            pltpu.make_async_copy(
                stage.at[k % 2], out.at[pl.ds(out_off(bo, io), OUT * C)],
                osem.at[k % 2]).wait()

        pltpu.make_async_copy(table.at[idxb.at[p]], taps.at[p], sem.at[p]).wait()
        for j in range(OUT):
            acc_lo = [None] * 8
            acc_hi = [None] * 8
            for t in range(16):
                wspl = plsc.load_gather(
                    wb, [jnp.full((16,), p, jnp.int32),
                         jnp.full((16,), 16 + j * 16 + t, jnp.int32)])
                r = j * 16 + t
                for c in range(8):
                    vi = plsc.bitcast(taps[p, r, pl.ds(c * 16, 16)], jnp.int32)
                    lo = plsc.bitcast(vi << 16, jnp.float32)
                    hi = plsc.bitcast(vi & jnp.int32(-65536), jnp.float32)
                    tlo = wspl * lo
                    thi = wspl * hi
                    acc_lo[c] = tlo if t == 0 else acc_lo[c] + tlo
                    acc_hi[c] = thi if t == 0 else acc_hi[c] + thi
            for c in range(8):
                stage[k % 2, pl.ds(j * C + c * 16, 16)] = acc_lo[c]
                stage[k % 2, pl.ds(j * C + CW + c * 16, 16)] = acc_hi[c]
        pltpu.async_copy(stage.at[k % 2], out.at[pl.ds(out_off(b, i), OUT * C)],
                         osem.at[k % 2])
        return (bn, inx)

    lax.fori_loop(0, NSTEP, body, (jnp.int32(0), jnp.int32(0)))

    for k in (NSTEP - 2, NSTEP - 1):
        b, i = k // OUT, k % OUT
        pltpu.make_async_copy(
            stage.at[k % 2], out.at[pl.ds(out_off(b, i), OUT * C)],
            osem.at[k % 2]).wait()


@jax.jit
def _pooler(table, boxes_t):
    mesh = plsc.VectorSubcoreMesh(core_axis_name="c", subcore_axis_name="s")
    f = functools.partial(
        pl.kernel,
        out_type=jax.ShapeDtypeStruct((NBOX_PAD * 49 * C,), jnp.float32),
        mesh=mesh,
        compiler_params=pltpu.CompilerParams(needs_layout_passes=False),
        scratch_types=[
            pltpu.VMEM((4 * BPW,), jnp.float32),
            pltpu.VMEM((16 + 6 * BPW,), jnp.float32),
            pltpu.VMEM((NSLOT, OUT * 16), jnp.int32),
            pltpu.VMEM((NSLOT, 16 + OUT * 16), jnp.float32),
            pltpu.VMEM((NSLOT, OUT * 16, CW), jnp.float32),
            pltpu.VMEM((2, OUT * C), jnp.float32),
            pltpu.SemaphoreType.DMA((NSLOT,)),
            pltpu.SemaphoreType.DMA((2,)),
        ],
    )(_sc_body)
    return f(table, boxes_t)


def kernel(feat0, feat1, feat2, feat3, boxes):
    feats = (feat0[0], feat1[0], feat2[0], feat3[0])
    table = jnp.concatenate(
        [jnp.transpose(f, (1, 2, 0)).reshape(-1, C) for f in feats], axis=0)
    packed = lax.bitcast_convert_type(
        table.astype(jnp.bfloat16).reshape(TABLE_ROWS, CW, 2), jnp.float32)
    boxes_t = jnp.zeros((4, NBOX_PAD), jnp.float32).at[:, :NBOX].set(boxes.T).reshape(-1)
    out = _pooler(packed, boxes_t)
    res = out.reshape(NBOX_PAD, 49, 2, CW)[:NBOX]
    res = jnp.transpose(res, (0, 3, 2, 1)).reshape(NBOX, C, 49)
    return res.reshape(NBOX, C, OUT, OUT)
